# baseline (speedup 1.0000x reference)
import dataclasses
import functools

import jax
import jax.numpy as jnp
from jax import lax
from jax.experimental import pallas as pl
from jax.experimental.pallas import tpu as pltpu
from jax.experimental.pallas import tpu_sc as plsc

N = 10000
E = 320000
D = 128
NUM_LAYERS = 3

NC = 2
NS = 16
NW = NC * NS
CHUNK = 128
NCHUNK0 = 64
NCHUNK1 = 96
E_PAD = NS * (NCHUNK0 + NCHUNK1) * CHUNK
PKW = CHUNK * 5 + 16
TRASH = N
N_ACC = 10240
RPT = N_ACC // NS

_PREC = lax.Precision.HIGHEST


def _sc_edge_stage(compute_deg):
  mesh = plsc.VectorSubcoreMesh(core_axis_name="c", subcore_axis_name="s")
  out_type = [jax.ShapeDtypeStruct((NC, N_ACC, D), jnp.float32)]
  if compute_deg:
    out_type.append(jax.ShapeDtypeStruct((NC, N_ACC), jnp.float32))
  scratch = [
      pltpu.VMEM((CHUNK,), jnp.int32),
      pltpu.VMEM((CHUNK,), jnp.int32),
      pltpu.VMEM((CHUNK * 4 + 16,), jnp.float32),
      pltpu.VMEM((CHUNK * 4 + 16,), jnp.float32),
      pltpu.VMEM((CHUNK,), jnp.int32),
      pltpu.VMEM((CHUNK,), jnp.int32),
      pltpu.VMEM((CHUNK, D), jnp.float32),
      pltpu.VMEM((CHUNK, D), jnp.float32),
      pltpu.VMEM((4, D), jnp.float32),
      pltpu.VMEM_SHARED((N_ACC, D), jnp.float32),
      pltpu.SemaphoreType.DMA,
      pltpu.SemaphoreType.DMA,
      pltpu.SemaphoreType.DMA,
      pltpu.SemaphoreType.DMA,
      pltpu.SemaphoreType.DMA,
      pltpu.SemaphoreType.DMA,
  ]
  if compute_deg:
    scratch += [
        pltpu.VMEM((CHUNK,), jnp.float32),
        pltpu.VMEM((RPT,), jnp.float32),
        pltpu.VMEM_SHARED((N_ACC,), jnp.float32),
    ]

  def body(hx_hbm, src_hbm, ea_hbm, dst_hbm, w0e_hbm, *refs):
    if compute_deg:
      (s_out, deg_out, si0, si1, ea0, ea1, di0, di1, rows0, rows1, w0ev, acc,
       spk0, spk1, sdi0, sdi1, sg0, sg1, onesv, degb, dacc) = refs
    else:
      (s_out, si0, si1, ea0, ea1, di0, di1, rows0, rows1, w0ev, acc,
       spk0, spk1, sdi0, sdi1, sg0, sg1) = refs
    si = (si0, si1)
    ea = (ea0, ea1)
    di = (di0, di1)
    rows = (rows0, rows1)
    spk = (spk0, spk1)
    sdi = (sdi0, sdi1)
    sg = (sg0, sg1)

    cid = lax.axis_index("c")
    sid = lax.axis_index("s")
    tb = sid * RPT
    nch = jnp.where(cid == 0, NCHUNK0, NCHUNK1)
    base_edges = jnp.where(cid == 0, sid * NCHUNK0 * CHUNK,
                           NS * NCHUNK0 * CHUNK + sid * NCHUNK1 * CHUNK)

    pltpu.sync_copy(w0e_hbm, w0ev)

    zeros16 = jnp.zeros((16,), jnp.float32)

    @pl.loop(0, CHUNK)
    def _(r):
      for j in range(D // 16):
        rows0[r, pl.ds(j * 16, 16)] = zeros16

    for p in range(RPT // CHUNK):
      pltpu.sync_copy(rows0, acc.at[pl.ds(tb + p * CHUNK, CHUNK)])
    if compute_deg:
      @pl.loop(0, RPT, step=16)
      def _(r):
        degb[pl.ds(r, 16)] = zeros16

      pltpu.sync_copy(degb, dacc.at[pl.ds(tb, RPT)])

      @pl.loop(0, CHUNK, step=16)
      def _(r):
        onesv[pl.ds(r, 16)] = jnp.ones((16,), jnp.float32)

    plsc.subcore_barrier()

    base0 = base_edges

    def issue_idx(c, b):
      pltpu.async_copy(src_hbm.at[pl.ds(base0 + c * CHUNK, CHUNK)],
                       si[b], spk[b])
      pltpu.async_copy(ea_hbm.at[pl.ds((base0 + c * CHUNK) * 4, CHUNK * 4)],
                       ea[b].at[pl.ds(0, CHUNK * 4)], spk[b])
      pltpu.async_copy(dst_hbm.at[pl.ds(base0 + c * CHUNK, CHUNK)],
                       di[b], sdi[b])

    def wait_idx_pk(c, b):
      pltpu.make_async_copy(src_hbm.at[pl.ds(base0 + c * CHUNK, CHUNK)],
                            si[b], spk[b]).wait()
      pltpu.make_async_copy(ea_hbm.at[pl.ds((base0 + c * CHUNK) * 4,
                                            CHUNK * 4)],
                            ea[b].at[pl.ds(0, CHUNK * 4)], spk[b]).wait()

    def wait_idx_di(c, b):
      pltpu.make_async_copy(dst_hbm.at[pl.ds(base0 + c * CHUNK, CHUNK)],
                            di[b], sdi[b]).wait()

    def issue_gather(b):
      pltpu.async_copy(hx_hbm.at[si[b]], rows[b], sg[b])

    def wait_gather(b):
      pltpu.make_async_copy(hx_hbm.at[si[b]], rows[b], sg[b]).wait()

    issue_idx(0, 0)
    issue_idx(1, 1)
    wait_idx_pk(0, 0)
    issue_gather(0)

    wvecs_outer = [[w0ev[k, pl.ds(j * 16, 16)] for k in range(4)]
                   for j in range(D // 16)]

    def compute(b):
      rb = rows[b]
      eab = ea[b]

      @plsc.parallel_loop(0, CHUNK, unroll=2)
      def _(e):
        av = eab[pl.ds(e * 4, 16)]
        a0 = av[0]
        a1 = av[1]
        a2 = av[2]
        a3 = av[3]
        for j in range(D // 16):
          sl = pl.ds(j * 16, 16)
          w = wvecs_outer[j]
          t01 = a0 * w[0] + a1 * w[1]
          t23 = a2 * w[2] + a3 * w[3]
          z = (rb[e, sl] + t01) + t23
          rb[e, sl] = jnp.maximum(z, 0.0)

    def scatter(cc, b):
      wait_idx_di(cc, b)
      pltpu.sync_copy(rows[b], acc.at[di[b]], add=True)
      if compute_deg:
        pltpu.sync_copy(onesv, dacc.at[di[b]], add=True)

    @pl.loop(0, nch - 2, step=2)
    def _(c):
      for b in range(2):
        cc = c + b
        wait_gather(b)
        wait_idx_pk(cc + 1, 1 - b)
        issue_gather(1 - b)
        compute(b)
        scatter(cc, b)
        issue_idx(cc + 2, b)

    wait_gather(0)
    wait_idx_pk(nch - 1, 1)
    issue_gather(1)
    compute(0)
    scatter(nch - 2, 0)
    wait_gather(1)
    compute(1)
    scatter(nch - 1, 1)

    plsc.subcore_barrier()

    for p in range(RPT // CHUNK):
      pltpu.sync_copy(acc.at[pl.ds(tb + p * CHUNK, CHUNK)], rows0)
      pltpu.sync_copy(rows0, s_out.at[cid, pl.ds(tb + p * CHUNK, CHUNK)])
    if compute_deg:
      pltpu.sync_copy(dacc.at[pl.ds(tb, RPT)], degb)
      pltpu.sync_copy(degb, deg_out.at[cid, pl.ds(tb, RPT)])

  cp = pltpu.CompilerParams()
  if "needs_layout_passes" in pltpu.CompilerParams.__dataclass_fields__:
    cp = dataclasses.replace(cp, needs_layout_passes=False)
  return pl.kernel(body, out_type=out_type, mesh=mesh, scratch_types=scratch,
                   compiler_params=cp)


_ROWS_BLK = 1000
_GRID = N // _ROWS_BLK


def _rows_spec():
  return pl.BlockSpec((_ROWS_BLK, D), lambda i: (i, 0))


def _w_spec():
  return pl.BlockSpec((D, D), lambda i: (0, 0))


def _b_spec():
  return pl.BlockSpec((1, D), lambda i: (0, 0))


def _dot(a, b):
  return jnp.dot(a, b, preferred_element_type=jnp.float32, precision=_PREC)


def _tc_input_stage(x, wint, binb, w0xt, b0b):
  def body(x_ref, wi_ref, bi_ref, wx_ref, b0_ref, h_ref, hx_ref):
    h = _dot(x_ref[...], wi_ref[...]) + bi_ref[...]
    h_ref[...] = h
    hx_ref[...] = _dot(h, wx_ref[...]) + b0_ref[...]

  return pl.pallas_call(
      body,
      grid=(_GRID,),
      in_specs=[_rows_spec(), _w_spec(), _b_spec(), _w_spec(), _b_spec()],
      out_specs=[_rows_spec(), _rows_spec()],
      out_shape=[
          jax.ShapeDtypeStruct((N, D), jnp.float32),
          jax.ShapeDtypeStruct((N, D), jnp.float32),
      ],
  )(x, wint, binb, w0xt, b0b)


def _tc_layer_stage(h, s0, s1, d0, d1, w1t, b1b, wu0at, wu0bt, bu0b, wu1t,
                    bu1b, gb, betab, wnt, bnb):
  def body(h_ref, s0_ref, s1_ref, d0_ref, d1_ref, w1_ref, b1_ref, wa_ref,
           wb_ref, bu0_ref, wu1_ref, bu1_ref, g_ref, be_ref, wn_ref, bn_ref,
           hn_ref, hxn_ref):
    s = s0_ref[...] + s1_ref[...]
    deg = d0_ref[...] + d1_ref[...]
    aggr = _dot(s, w1_ref[...]) + deg * b1_ref[...]
    h = h_ref[...]
    t = _dot(h, wa_ref[...]) + _dot(aggr, wb_ref[...]) + bu0_ref[...]
    u = _dot(jnp.maximum(t, 0.0), wu1_ref[...]) + bu1_ref[...]
    hn = h + u
    m = jnp.mean(hn, axis=-1, keepdims=True)
    cdev = hn - m
    v = jnp.mean(cdev * cdev, axis=-1, keepdims=True)
    hn = cdev / jnp.sqrt(v + 1e-5) * g_ref[...] + be_ref[...]
    hn_ref[...] = hn
    hxn_ref[...] = _dot(hn, wn_ref[...]) + bn_ref[...]

  dspec = pl.BlockSpec((_ROWS_BLK, 1), lambda i: (i, 0))
  return pl.pallas_call(
      body,
      grid=(_GRID,),
      in_specs=[
          _rows_spec(), _rows_spec(), _rows_spec(), dspec, dspec,
          _w_spec(), _b_spec(), _w_spec(), _w_spec(), _b_spec(), _w_spec(),
          _b_spec(), _b_spec(), _b_spec(), _w_spec(), _b_spec(),
      ],
      out_specs=[_rows_spec(), _rows_spec()],
      out_shape=[
          jax.ShapeDtypeStruct((N, D), jnp.float32),
          jax.ShapeDtypeStruct((N, D), jnp.float32),
      ],
  )(h, s0, s1, d0, d1, w1t, b1b, wu0at, wu0bt, bu0b, wu1t, bu1b, gb, betab,
    wnt, bnb)


@jax.jit
def kernel(x, edge_index, edge_attr, params):
  src = edge_index[0].astype(jnp.int32)
  dst = edge_index[1].astype(jnp.int32)
  pad = E_PAD - E
  src_p = jnp.concatenate([src, jnp.zeros((pad,), jnp.int32)])
  dst_p = jnp.concatenate([dst, jnp.full((pad,), TRASH, jnp.int32)])
  ea_p = jnp.concatenate([edge_attr.reshape(E * 4),
                          jnp.zeros((pad * 4,), jnp.float32)])

  convs = params["convs"]
  w0 = [convs[i]["message_mlp"]["l0"]["w"] for i in range(NUM_LAYERS)]
  w0xt = [w[:, :D].T for w in w0]
  w0e = [w[:, D:].T.reshape(4, D) for w in w0]
  b0 = [convs[i]["message_mlp"]["l0"]["b"] for i in range(NUM_LAYERS)]

  h, hx = _tc_input_stage(
      x, params["input_proj"]["w"].T,
      params["input_proj"]["b"].reshape(1, D), w0xt[0],
      b0[0].reshape(1, D))

  sc0 = _sc_edge_stage(True)
  sc = _sc_edge_stage(False)
  d0 = d1 = None
  for i in range(NUM_LAYERS):
    if i == 0:
      s_acc, degs = sc0(hx, src_p, ea_p, dst_p, w0e[i])
      d0 = degs[0, :N].reshape(N, 1)
      d1 = degs[1, :N].reshape(N, 1)
    else:
      (s_acc,) = sc(hx, src_p, ea_p, dst_p, w0e[i])
    c = convs[i]
    if i + 1 < NUM_LAYERS:
      wnt = w0xt[i + 1]
      bnb = b0[i + 1].reshape(1, D)
    else:
      wnt = params["output_proj"]["w"].T
      bnb = params["output_proj"]["b"].reshape(1, D)
    wu0 = c["update_mlp"]["l0"]["w"]
    h, hx = _tc_layer_stage(
        h, s_acc[0, :N], s_acc[1, :N], d0, d1,
        c["message_mlp"]["l1"]["w"].T,
        c["message_mlp"]["l1"]["b"].reshape(1, D),
        wu0[:, :D].T, wu0[:, D:].T,
        c["update_mlp"]["l0"]["b"].reshape(1, D),
        c["update_mlp"]["l1"]["w"].T,
        c["update_mlp"]["l1"]["b"].reshape(1, D),
        params["lns"][i]["g"].reshape(1, D),
        params["lns"][i]["b"].reshape(1, D),
        wnt, bnb)
  return hx


# speedup vs baseline: 1.1834x; 1.1834x over previous
import dataclasses
import functools

import jax
import jax.numpy as jnp
from jax import lax
from jax.experimental import pallas as pl
from jax.experimental.pallas import tpu as pltpu
from jax.experimental.pallas import tpu_sc as plsc

N = 10000
E = 320000
D = 128
NUM_LAYERS = 3

NC = 2
NS = 16
NW = NC * NS
CHUNK = 128
NCHUNK0 = 96
NCHUNK1 = 64
E_PAD = NS * (NCHUNK0 + NCHUNK1) * CHUNK
PKW = CHUNK * 5 + 16
TRASH = N
N_ACC = 10240
RPT = N_ACC // NS

_PREC = lax.Precision.HIGHEST


def _sc_edge_stage(compute_deg):
  mesh = plsc.VectorSubcoreMesh(core_axis_name="c", subcore_axis_name="s")
  out_type = [jax.ShapeDtypeStruct((NC, N_ACC, D), jnp.float32)]
  if compute_deg:
    out_type.append(jax.ShapeDtypeStruct((NC, N_ACC), jnp.float32))
  scratch = [
      pltpu.VMEM((CHUNK,), jnp.int32),
      pltpu.VMEM((CHUNK,), jnp.int32),
      pltpu.VMEM((CHUNK * 4 + 16,), jnp.float32),
      pltpu.VMEM((CHUNK * 4 + 16,), jnp.float32),
      pltpu.VMEM((CHUNK,), jnp.int32),
      pltpu.VMEM((CHUNK,), jnp.int32),
      pltpu.VMEM((CHUNK, D), jnp.float32),
      pltpu.VMEM((CHUNK, D), jnp.float32),
      pltpu.VMEM((4, D), jnp.float32),
      pltpu.VMEM_SHARED((N_ACC, D), jnp.float32),
      pltpu.SemaphoreType.DMA,
      pltpu.SemaphoreType.DMA,
      pltpu.SemaphoreType.DMA,
      pltpu.SemaphoreType.DMA,
      pltpu.SemaphoreType.DMA,
      pltpu.SemaphoreType.DMA,
  ]
  if compute_deg:
    scratch += [
        pltpu.VMEM((CHUNK,), jnp.float32),
        pltpu.VMEM((RPT,), jnp.float32),
        pltpu.VMEM_SHARED((N_ACC,), jnp.float32),
    ]

  def body(hx_hbm, src_hbm, ea_hbm, dst_hbm, w0e_hbm, *refs):
    if compute_deg:
      (s_out, deg_out, si0, si1, ea0, ea1, di0, di1, rows0, rows1, w0ev, acc,
       spk0, spk1, sdi0, sdi1, sg0, sg1, onesv, degb, dacc) = refs
    else:
      (s_out, si0, si1, ea0, ea1, di0, di1, rows0, rows1, w0ev, acc,
       spk0, spk1, sdi0, sdi1, sg0, sg1) = refs
    si = (si0, si1)
    ea = (ea0, ea1)
    di = (di0, di1)
    rows = (rows0, rows1)
    spk = (spk0, spk1)
    sdi = (sdi0, sdi1)
    sg = (sg0, sg1)

    cid = lax.axis_index("c")
    sid = lax.axis_index("s")
    tb = sid * RPT
    nch = jnp.where(cid == 0, NCHUNK0, NCHUNK1)
    base_edges = jnp.where(cid == 0, sid * NCHUNK0 * CHUNK,
                           NS * NCHUNK0 * CHUNK + sid * NCHUNK1 * CHUNK)

    pltpu.sync_copy(w0e_hbm, w0ev)

    zeros16 = jnp.zeros((16,), jnp.float32)

    @pl.loop(0, CHUNK)
    def _(r):
      for j in range(D // 16):
        rows0[r, pl.ds(j * 16, 16)] = zeros16

    for p in range(RPT // CHUNK):
      pltpu.sync_copy(rows0, acc.at[pl.ds(tb + p * CHUNK, CHUNK)])
    if compute_deg:
      @pl.loop(0, RPT, step=16)
      def _(r):
        degb[pl.ds(r, 16)] = zeros16

      pltpu.sync_copy(degb, dacc.at[pl.ds(tb, RPT)])

      @pl.loop(0, CHUNK, step=16)
      def _(r):
        onesv[pl.ds(r, 16)] = jnp.ones((16,), jnp.float32)

    plsc.subcore_barrier()

    base0 = base_edges

    def issue_idx(c, b):
      pltpu.async_copy(src_hbm.at[pl.ds(base0 + c * CHUNK, CHUNK)],
                       si[b], spk[b])
      pltpu.async_copy(ea_hbm.at[pl.ds((base0 + c * CHUNK) * 4, CHUNK * 4)],
                       ea[b].at[pl.ds(0, CHUNK * 4)], spk[b])
      pltpu.async_copy(dst_hbm.at[pl.ds(base0 + c * CHUNK, CHUNK)],
                       di[b], sdi[b])

    def wait_idx_pk(c, b):
      pltpu.make_async_copy(src_hbm.at[pl.ds(base0 + c * CHUNK, CHUNK)],
                            si[b], spk[b]).wait()
      pltpu.make_async_copy(ea_hbm.at[pl.ds((base0 + c * CHUNK) * 4,
                                            CHUNK * 4)],
                            ea[b].at[pl.ds(0, CHUNK * 4)], spk[b]).wait()

    def wait_idx_di(c, b):
      pltpu.make_async_copy(dst_hbm.at[pl.ds(base0 + c * CHUNK, CHUNK)],
                            di[b], sdi[b]).wait()

    def issue_gather(b):
      pltpu.async_copy(hx_hbm.at[si[b]], rows[b], sg[b])

    def wait_gather(b):
      pltpu.make_async_copy(hx_hbm.at[si[b]], rows[b], sg[b]).wait()

    issue_idx(0, 0)
    issue_idx(1, 1)
    wait_idx_pk(0, 0)
    issue_gather(0)

    wvecs_outer = [[w0ev[k, pl.ds(j * 16, 16)] for k in range(4)]
                   for j in range(D // 16)]

    def compute(b):
      rb = rows[b]
      eab = ea[b]

      @plsc.parallel_loop(0, CHUNK, unroll=2)
      def _(e):
        av = eab[pl.ds(e * 4, 16)]
        a0 = av[0]
        a1 = av[1]
        a2 = av[2]
        a3 = av[3]
        for j in range(D // 16):
          sl = pl.ds(j * 16, 16)
          w = wvecs_outer[j]
          t01 = a0 * w[0] + a1 * w[1]
          t23 = a2 * w[2] + a3 * w[3]
          z = (rb[e, sl] + t01) + t23
          rb[e, sl] = jnp.maximum(z, 0.0)

    def scatter(cc, b):
      wait_idx_di(cc, b)
      pltpu.sync_copy(rows[b], acc.at[di[b]], add=True)
      if compute_deg:
        pltpu.sync_copy(onesv, dacc.at[di[b]], add=True)

    @pl.loop(0, nch - 2, step=2)
    def _(c):
      for b in range(2):
        cc = c + b
        wait_gather(b)
        wait_idx_pk(cc + 1, 1 - b)
        issue_gather(1 - b)
        compute(b)
        scatter(cc, b)
        issue_idx(cc + 2, b)

    wait_gather(0)
    wait_idx_pk(nch - 1, 1)
    issue_gather(1)
    compute(0)
    scatter(nch - 2, 0)
    wait_gather(1)
    compute(1)
    scatter(nch - 1, 1)

    plsc.subcore_barrier()

    for p in range(RPT // CHUNK):
      pltpu.sync_copy(acc.at[pl.ds(tb + p * CHUNK, CHUNK)], rows0)
      pltpu.sync_copy(rows0, s_out.at[cid, pl.ds(tb + p * CHUNK, CHUNK)])
    if compute_deg:
      pltpu.sync_copy(dacc.at[pl.ds(tb, RPT)], degb)
      pltpu.sync_copy(degb, deg_out.at[cid, pl.ds(tb, RPT)])

  cp = pltpu.CompilerParams()
  if "needs_layout_passes" in pltpu.CompilerParams.__dataclass_fields__:
    cp = dataclasses.replace(cp, needs_layout_passes=False)
  return pl.kernel(body, out_type=out_type, mesh=mesh, scratch_types=scratch,
                   compiler_params=cp)


_ROWS_BLK = 1000
_GRID = N // _ROWS_BLK


def _rows_spec():
  return pl.BlockSpec((_ROWS_BLK, D), lambda i: (i, 0))


def _w_spec():
  return pl.BlockSpec((D, D), lambda i: (0, 0))


def _b_spec():
  return pl.BlockSpec((1, D), lambda i: (0, 0))


def _dot(a, b):
  return jnp.dot(a, b, preferred_element_type=jnp.float32, precision=_PREC)


def _tc_input_stage(x, wint, binb, w0xt, b0b):
  def body(x_ref, wi_ref, bi_ref, wx_ref, b0_ref, h_ref, hx_ref):
    h = _dot(x_ref[...], wi_ref[...]) + bi_ref[...]
    h_ref[...] = h
    hx_ref[...] = _dot(h, wx_ref[...]) + b0_ref[...]

  return pl.pallas_call(
      body,
      grid=(_GRID,),
      in_specs=[_rows_spec(), _w_spec(), _b_spec(), _w_spec(), _b_spec()],
      out_specs=[_rows_spec(), _rows_spec()],
      out_shape=[
          jax.ShapeDtypeStruct((N, D), jnp.float32),
          jax.ShapeDtypeStruct((N, D), jnp.float32),
      ],
  )(x, wint, binb, w0xt, b0b)


def _tc_layer_stage(h, s0, s1, d0, d1, w1t, b1b, wu0at, wu0bt, bu0b, wu1t,
                    bu1b, gb, betab, wnt, bnb):
  def body(h_ref, s0_ref, s1_ref, d0_ref, d1_ref, w1_ref, b1_ref, wa_ref,
           wb_ref, bu0_ref, wu1_ref, bu1_ref, g_ref, be_ref, wn_ref, bn_ref,
           hn_ref, hxn_ref):
    s = s0_ref[...] + s1_ref[...]
    deg = d0_ref[...] + d1_ref[...]
    aggr = _dot(s, w1_ref[...]) + deg * b1_ref[...]
    h = h_ref[...]
    t = _dot(h, wa_ref[...]) + _dot(aggr, wb_ref[...]) + bu0_ref[...]
    u = _dot(jnp.maximum(t, 0.0), wu1_ref[...]) + bu1_ref[...]
    hn = h + u
    m = jnp.mean(hn, axis=-1, keepdims=True)
    cdev = hn - m
    v = jnp.mean(cdev * cdev, axis=-1, keepdims=True)
    hn = cdev / jnp.sqrt(v + 1e-5) * g_ref[...] + be_ref[...]
    hn_ref[...] = hn
    hxn_ref[...] = _dot(hn, wn_ref[...]) + bn_ref[...]

  dspec = pl.BlockSpec((_ROWS_BLK, 1), lambda i: (i, 0))
  return pl.pallas_call(
      body,
      grid=(_GRID,),
      in_specs=[
          _rows_spec(), _rows_spec(), _rows_spec(), dspec, dspec,
          _w_spec(), _b_spec(), _w_spec(), _w_spec(), _b_spec(), _w_spec(),
          _b_spec(), _b_spec(), _b_spec(), _w_spec(), _b_spec(),
      ],
      out_specs=[_rows_spec(), _rows_spec()],
      out_shape=[
          jax.ShapeDtypeStruct((N, D), jnp.float32),
          jax.ShapeDtypeStruct((N, D), jnp.float32),
      ],
  )(h, s0, s1, d0, d1, w1t, b1b, wu0at, wu0bt, bu0b, wu1t, bu1b, gb, betab,
    wnt, bnb)


@jax.jit
def kernel(x, edge_index, edge_attr, params):
  src = edge_index[0].astype(jnp.int32)
  dst = edge_index[1].astype(jnp.int32)
  pad = E_PAD - E
  src_p = jnp.concatenate([src, jnp.zeros((pad,), jnp.int32)])
  dst_p = jnp.concatenate([dst, jnp.full((pad,), TRASH, jnp.int32)])
  ea_p = jnp.concatenate([edge_attr.reshape(E * 4),
                          jnp.zeros((pad * 4,), jnp.float32)])

  convs = params["convs"]
  w0 = [convs[i]["message_mlp"]["l0"]["w"] for i in range(NUM_LAYERS)]
  w0xt = [w[:, :D].T for w in w0]
  w0e = [w[:, D:].T.reshape(4, D) for w in w0]
  b0 = [convs[i]["message_mlp"]["l0"]["b"] for i in range(NUM_LAYERS)]

  h, hx = _tc_input_stage(
      x, params["input_proj"]["w"].T,
      params["input_proj"]["b"].reshape(1, D), w0xt[0],
      b0[0].reshape(1, D))

  sc0 = _sc_edge_stage(True)
  sc = _sc_edge_stage(False)
  d0 = d1 = None
  for i in range(NUM_LAYERS):
    if i == 0:
      s_acc, degs = sc0(hx, src_p, ea_p, dst_p, w0e[i])
      d0 = degs[0, :N].reshape(N, 1)
      d1 = degs[1, :N].reshape(N, 1)
    else:
      (s_acc,) = sc(hx, src_p, ea_p, dst_p, w0e[i])
    c = convs[i]
    if i + 1 < NUM_LAYERS:
      wnt = w0xt[i + 1]
      bnb = b0[i + 1].reshape(1, D)
    else:
      wnt = params["output_proj"]["w"].T
      bnb = params["output_proj"]["b"].reshape(1, D)
    wu0 = c["update_mlp"]["l0"]["w"]
    h, hx = _tc_layer_stage(
        h, s_acc[0, :N], s_acc[1, :N], d0, d1,
        c["message_mlp"]["l1"]["w"].T,
        c["message_mlp"]["l1"]["b"].reshape(1, D),
        wu0[:, :D].T, wu0[:, D:].T,
        c["update_mlp"]["l0"]["b"].reshape(1, D),
        c["update_mlp"]["l1"]["w"].T,
        c["update_mlp"]["l1"]["b"].reshape(1, D),
        params["lns"][i]["g"].reshape(1, D),
        params["lns"][i]["b"].reshape(1, D),
        wnt, bnb)
  return hx
